# baseline (speedup 1.0000x reference)
import functools

import jax
import jax.numpy as jnp
from jax import lax
from jax.experimental import pallas as pl
from jax.experimental.pallas import tpu as pltpu
from jax.experimental.pallas import tpu_sc as plsc

T = 8
N = 10000
E = 320000
D = 128
H = 128
M = 64
TW = 10

NC = 2
NS = 16
CHUNK = 128
EPT = 10240
EPAD = NC * NS * EPT
CPT = EPT // CHUNK
NPAD = 10240
NPS = NPAD // NS
ZCH = 40


def _s_body(x_ref, w_ref, b_ref, o_ref):
    o_ref[...] = jnp.maximum(
        jnp.dot(x_ref[...], w_ref[...], preferred_element_type=jnp.float32)
        + b_ref[...], 0.0)


def _compute_s(x_flat, w, b):
    grid = 40
    bn = (T * N) // grid
    return pl.pallas_call(
        _s_body,
        grid=(grid,),
        in_specs=[
            pl.BlockSpec((bn, D), lambda i: (i, 0)),
            pl.BlockSpec((D, H), lambda i: (0, 0)),
            pl.BlockSpec((1, H), lambda i: (0, 0)),
        ],
        out_specs=pl.BlockSpec((bn, H), lambda i: (i, 0)),
        out_shape=jax.ShapeDtypeStruct((T * N, H), jnp.float32),
    )(x_flat, w, b.reshape(1, H))


IB = 8
NBLK = CPT // IB
NSLOT = 4


def _sc_body(s_hbm, idx_hbm, zeros_hbm, out_hbm,
             idx0, idx1, idx2, idx3, rows0, rows1, zrows, msgs_sh,
             gsem0, gsem1, gsem2, gsem3, gsem4, gsem5, gsem6, gsem7,
             isem0, isem1, isem2, isem3):
    c = lax.axis_index("c")
    s = lax.axis_index("s")
    wid = c * NS + s
    idxb = (idx0, idx1, idx2, idx3)
    isem = (isem0, isem1, isem2, isem3)
    rows = (rows0, rows1)
    gsem = (gsem0, gsem1, gsem2, gsem3, gsem4, gsem5, gsem6, gsem7)
    pltpu.sync_copy(zeros_hbm, zrows)

    def body(t, carry):
        st = s_hbm.at[t]

        def load_idx(b):
            return pltpu.async_copy(
                idx_hbm.at[wid, pl.ds(b * IB, IB)], idxb[b % NSLOT],
                isem[b % NSLOT])

        def gather(j, h):
            b, r = divmod(j, IB)
            return pltpu.async_copy(
                st.at[idxb[b % NSLOT].at[r, 0, pl.ds(h * 32, 32)]],
                rows[j % 2].at[pl.ds(h * 32, 32)], gsem[4 * (j % 2) + h])

        idx_cp = {b: load_idx(b) for b in range(NSLOT)}

        for k in range(NPS // ZCH):
            pltpu.sync_copy(zrows, msgs_sh.at[pl.ds(s * NPS + k * ZCH, ZCH)])
        plsc.subcore_barrier()

        idx_cp[0].wait()
        g_cp = {(j0, h): gather(j0, h) for j0 in (0, 1) for h in range(4)}
        waited = {0: True}
        for j in range(NBLK * IB):
            b, r = divmod(j, IB)
            if b not in waited:
                idx_cp[b].wait()
                waited[b] = True
            for h in range(4):
                g_cp[(j, h)].wait()
            pltpu.sync_copy(rows[j % 2], msgs_sh.at[idxb[b % NSLOT].at[r, 1]],
                            add=True)
            if j + 2 < NBLK * IB:
                for h in range(4):
                    g_cp[(j + 2, h)] = gather(j + 2, h)
            if r == IB - 1 and b + NSLOT < NBLK:
                idx_cp[b + NSLOT] = load_idx(b + NSLOT)
        plsc.subcore_barrier()
        pltpu.sync_copy(msgs_sh.at[pl.ds(s * NPS, NPS)],
                        out_hbm.at[c, t, pl.ds(s * NPS, NPS)])
        plsc.subcore_barrier()
        return carry

    lax.fori_loop(0, T, body, 0)


def _compute_msgs(s3, idx4d, zeros_block):
    mesh = plsc.VectorSubcoreMesh(core_axis_name="c", subcore_axis_name="s")
    fn = functools.partial(
        pl.kernel,
        out_type=jax.ShapeDtypeStruct((NC, T, NPAD, H), jnp.float32),
        mesh=mesh,
        scratch_types=[
            pltpu.VMEM((IB, 2, CHUNK), jnp.int32),
            pltpu.VMEM((IB, 2, CHUNK), jnp.int32),
            pltpu.VMEM((IB, 2, CHUNK), jnp.int32),
            pltpu.VMEM((IB, 2, CHUNK), jnp.int32),
            pltpu.VMEM((CHUNK, H), jnp.float32),
            pltpu.VMEM((CHUNK, H), jnp.float32),
            pltpu.VMEM((ZCH, H), jnp.float32),
            pltpu.VMEM_SHARED((NPAD, H), jnp.float32),
        ] + [pltpu.SemaphoreType.DMA] * 12,
    )(_sc_body)
    return fn(s3, idx4d, zeros_block)
def _gru_body(s_ref, p_ref, p1_ref, p2_ref, ba_ref, wh_ref, bhn_ref,
              wos_ref, wom_ref, bf_ref, final_ref, mem_ref):
    bn = s_ref.shape[1]
    mem = jnp.zeros((bn, M), jnp.float32)
    for t in range(T):
        msum = p_ref[0, t] + p_ref[1, t]
        a = (jnp.dot(s_ref[t], p1_ref[...], preferred_element_type=jnp.float32)
             + jnp.dot(msum, p2_ref[...], preferred_element_type=jnp.float32)
             + ba_ref[...])
        g = jnp.dot(mem, wh_ref[...], preferred_element_type=jnp.float32)
        r = jax.nn.sigmoid(a[:, :M] + g[:, :M])
        z = jax.nn.sigmoid(a[:, M:2 * M] + g[:, M:2 * M])
        n = jnp.tanh(a[:, 2 * M:] + r * (g[:, 2 * M:] + bhn_ref[...]))
        mem = (1.0 - z) * n + z * mem
    final_ref[...] = (
        jnp.dot(s_ref[T - 1], wos_ref[...], preferred_element_type=jnp.float32)
        + jnp.dot(mem, wom_ref[...], preferred_element_type=jnp.float32)
        + bf_ref[...])
    mem_ref[...] = mem


def _compute_gru(s3, p, p1, p2, ba, wh, bhn, wos, wom, bf):
    grid = 10
    bn = N // grid
    return pl.pallas_call(
        _gru_body,
        grid=(grid,),
        in_specs=[
            pl.BlockSpec((T, bn, H), lambda i: (0, i, 0)),
            pl.BlockSpec((NC, T, bn, H), lambda i: (0, 0, i, 0)),
            pl.BlockSpec((H, 3 * M), lambda i: (0, 0)),
            pl.BlockSpec((H, 3 * M), lambda i: (0, 0)),
            pl.BlockSpec((1, 3 * M), lambda i: (0, 0)),
            pl.BlockSpec((M, 3 * M), lambda i: (0, 0)),
            pl.BlockSpec((1, M), lambda i: (0, 0)),
            pl.BlockSpec((H, H), lambda i: (0, 0)),
            pl.BlockSpec((M, H), lambda i: (0, 0)),
            pl.BlockSpec((1, H), lambda i: (0, 0)),
        ],
        out_specs=[
            pl.BlockSpec((bn, H), lambda i: (i, 0)),
            pl.BlockSpec((bn, M), lambda i: (i, 0)),
        ],
        out_shape=[
            jax.ShapeDtypeStruct((N, H), jnp.float32),
            jax.ShapeDtypeStruct((N, M), jnp.float32),
        ],
    )(s3, p, p1, p2, ba, wh, bhn, wos, wom, bf)


def kernel(node_features, edge_index, W_s, b_s, W_mp, b_mp, Wi_r, bi_r, Wh_r,
           Wi_z, bi_z, Wh_z, Wi_n, bi_n, Wh_n, bh_n, W_o, b_o, conv_k, conv_b):
    Wi_cat = jnp.concatenate([Wi_r, Wi_z, Wi_n], axis=1)
    bi_cat = jnp.concatenate([bi_r, bi_z, bi_n])
    p1 = W_mp[:H] @ Wi_cat
    p2 = W_mp[H:] @ Wi_cat
    ba = (b_mp @ Wi_cat + bi_cat).reshape(1, 3 * M)
    wh = jnp.concatenate([Wh_r, Wh_z, Wh_n], axis=1)
    ii = jnp.arange(H)[:, None]
    hh = jnp.arange(H)[None, :]
    kk = ii - hh + (TW - 1) // 2
    cmat = jnp.where((kk >= 0) & (kk < TW),
                     conv_k[jnp.clip(kk, 0, TW - 1), 0, 0], 0.0)
    wos = W_o[:H] @ cmat
    wom = W_o[H:] @ cmat
    bf = (b_o @ cmat + conv_b[0]).reshape(1, H)

    s_flat = _compute_s(node_features.reshape(T * N, D), W_s, b_s)
    s3 = s_flat.reshape(T, N, H)
    pad = jnp.zeros((EPAD - E,), jnp.int32)
    src2d = jnp.concatenate([edge_index[0], pad]).reshape(NC * NS, CPT, 1, CHUNK)
    dst2d = jnp.concatenate([edge_index[1], pad + N]).reshape(NC * NS, CPT, 1, CHUNK)
    idx4d = jnp.concatenate([src2d, dst2d], axis=2)
    zeros_block = jnp.zeros((ZCH, H), jnp.float32)
    p = _compute_msgs(s3, idx4d, zeros_block)
    final, mem = _compute_gru(s3, p, p1, p2, ba, wh,
                              bh_n.reshape(1, M), wos, wom, bf)
    return (final, mem)


# speedup vs baseline: 1.0974x; 1.0974x over previous
import functools

import jax
import jax.numpy as jnp
from jax import lax
from jax.experimental import pallas as pl
from jax.experimental.pallas import tpu as pltpu
from jax.experimental.pallas import tpu_sc as plsc

T = 8
N = 10000
E = 320000
D = 128
H = 128
M = 64
TW = 10

NC = 2
NS = 16
CHUNK = 128
EPT = 10240
EPAD = NC * NS * EPT
CPT = EPT // CHUNK
NPAD = 10240
NPS = NPAD // NS
ZCH = 40


def _s_body(x_ref, w_ref, b_ref, o_ref):
    o_ref[...] = jnp.maximum(
        jnp.dot(x_ref[...], w_ref[...], preferred_element_type=jnp.float32)
        + b_ref[...], 0.0)


def _compute_s(x_flat, w, b):
    grid = 40
    bn = (T * N) // grid
    return pl.pallas_call(
        _s_body,
        grid=(grid,),
        in_specs=[
            pl.BlockSpec((bn, D), lambda i: (i, 0)),
            pl.BlockSpec((D, H), lambda i: (0, 0)),
            pl.BlockSpec((1, H), lambda i: (0, 0)),
        ],
        out_specs=pl.BlockSpec((bn, H), lambda i: (i, 0)),
        out_shape=jax.ShapeDtypeStruct((T * N, H), jnp.float32),
    )(x_flat, w, b.reshape(1, H))


IB = 8
NBLK = CPT // IB
NSLOT = 4


def _sc_body(s_hbm, idx_hbm, zeros_hbm, out0_hbm, out1_hbm,
             idx0, idx1, idx2, idx3, rows0, rows1, zrows, msgs_sh,
             gsem0, gsem1, isem0, isem1, isem2, isem3):
    c = lax.axis_index("c")
    s = lax.axis_index("s")
    wid = c * NS + s
    idxb = (idx0, idx1, idx2, idx3)
    isem = (isem0, isem1, isem2, isem3)
    rows = (rows0, rows1)
    gsem = (gsem0, gsem1)
    pltpu.sync_copy(zeros_hbm, zrows)

    def body(t, carry):
        st = s_hbm.at[t]

        def load_idx(b):
            return pltpu.async_copy(
                idx_hbm.at[wid, pl.ds(b * IB, IB)], idxb[b % NSLOT],
                isem[b % NSLOT])

        def gather(j):
            b, r = divmod(j, IB)
            return pltpu.async_copy(
                st.at[idxb[b % NSLOT].at[r, 0]], rows[j % 2], gsem[j % 2])

        idx_cp = {b: load_idx(b) for b in range(NSLOT)}

        for k in range(NPS // ZCH):
            pltpu.sync_copy(zrows, msgs_sh.at[pl.ds(s * NPS + k * ZCH, ZCH)])
        plsc.subcore_barrier()

        idx_cp[0].wait()
        g_cp = {0: gather(0), 1: gather(1)}
        waited = {0: True}
        for j in range(NBLK * IB):
            b, r = divmod(j, IB)
            if b not in waited:
                idx_cp[b].wait()
                waited[b] = True
            g_cp[j].wait()
            pltpu.sync_copy(rows[j % 2], msgs_sh.at[idxb[b % NSLOT].at[r, 1]],
                            add=True)
            if j + 2 < NBLK * IB:
                g_cp[j + 2] = gather(j + 2)
            if r == IB - 1 and b + NSLOT < NBLK:
                idx_cp[b + NSLOT] = load_idx(b + NSLOT)
        plsc.subcore_barrier()
        @pl.when(c == 0)
        def _():
            pltpu.sync_copy(msgs_sh.at[pl.ds(s * NPS, NPS)],
                            out0_hbm.at[t, pl.ds(s * NPS, NPS)])

        @pl.when(c == 1)
        def _():
            pltpu.sync_copy(msgs_sh.at[pl.ds(s * NPS, NPS)],
                            out1_hbm.at[t, pl.ds(s * NPS, NPS)])

        plsc.subcore_barrier()
        return carry

    lax.fori_loop(0, T, body, 0)


def _compute_msgs(s3, idx4d, zeros_block):
    mesh = plsc.VectorSubcoreMesh(core_axis_name="c", subcore_axis_name="s")
    fn = functools.partial(
        pl.kernel,
        out_type=[jax.ShapeDtypeStruct((T, NPAD, H), jnp.float32),
                  jax.ShapeDtypeStruct((T, NPAD, H), jnp.float32)],
        mesh=mesh,
        scratch_types=[
            pltpu.VMEM((IB, 2, CHUNK), jnp.int32),
            pltpu.VMEM((IB, 2, CHUNK), jnp.int32),
            pltpu.VMEM((IB, 2, CHUNK), jnp.int32),
            pltpu.VMEM((IB, 2, CHUNK), jnp.int32),
            pltpu.VMEM((CHUNK, H), jnp.float32),
            pltpu.VMEM((CHUNK, H), jnp.float32),
            pltpu.VMEM((ZCH, H), jnp.float32),
            pltpu.VMEM_SHARED((NPAD, H), jnp.float32),
            pltpu.SemaphoreType.DMA,
            pltpu.SemaphoreType.DMA,
            pltpu.SemaphoreType.DMA,
            pltpu.SemaphoreType.DMA,
            pltpu.SemaphoreType.DMA,
            pltpu.SemaphoreType.DMA,
        ],
    )(_sc_body)
    return fn(s3, idx4d, zeros_block)
def _gru_body(s_ref, pa_ref, pb_ref, p1_ref, p2_ref, ba_ref, wh_ref, bhn_ref,
              wos_ref, wom_ref, bf_ref, final_ref, mem_ref):
    bn = s_ref.shape[1]
    mem = jnp.zeros((bn, M), jnp.float32)
    for t in range(T):
        msum = pa_ref[t] + pb_ref[t]
        a = (jnp.dot(s_ref[t], p1_ref[...], preferred_element_type=jnp.float32)
             + jnp.dot(msum, p2_ref[...], preferred_element_type=jnp.float32)
             + ba_ref[...])
        g = jnp.dot(mem, wh_ref[...], preferred_element_type=jnp.float32)
        r = jax.nn.sigmoid(a[:, :M] + g[:, :M])
        z = jax.nn.sigmoid(a[:, M:2 * M] + g[:, M:2 * M])
        n = jnp.tanh(a[:, 2 * M:] + r * (g[:, 2 * M:] + bhn_ref[...]))
        mem = (1.0 - z) * n + z * mem
    final_ref[...] = (
        jnp.dot(s_ref[T - 1], wos_ref[...], preferred_element_type=jnp.float32)
        + jnp.dot(mem, wom_ref[...], preferred_element_type=jnp.float32)
        + bf_ref[...])
    mem_ref[...] = mem


def _compute_gru(s3, pa, pb, p1, p2, ba, wh, bhn, wos, wom, bf):
    grid = 10
    bn = N // grid
    return pl.pallas_call(
        _gru_body,
        grid=(grid,),
        in_specs=[
            pl.BlockSpec((T, bn, H), lambda i: (0, i, 0)),
            pl.BlockSpec((T, bn, H), lambda i: (0, i, 0)),
            pl.BlockSpec((T, bn, H), lambda i: (0, i, 0)),
            pl.BlockSpec((H, 3 * M), lambda i: (0, 0)),
            pl.BlockSpec((H, 3 * M), lambda i: (0, 0)),
            pl.BlockSpec((1, 3 * M), lambda i: (0, 0)),
            pl.BlockSpec((M, 3 * M), lambda i: (0, 0)),
            pl.BlockSpec((1, M), lambda i: (0, 0)),
            pl.BlockSpec((H, H), lambda i: (0, 0)),
            pl.BlockSpec((M, H), lambda i: (0, 0)),
            pl.BlockSpec((1, H), lambda i: (0, 0)),
        ],
        out_specs=[
            pl.BlockSpec((bn, H), lambda i: (i, 0)),
            pl.BlockSpec((bn, M), lambda i: (i, 0)),
        ],
        out_shape=[
            jax.ShapeDtypeStruct((N, H), jnp.float32),
            jax.ShapeDtypeStruct((N, M), jnp.float32),
        ],
    )(s3, pa, pb, p1, p2, ba, wh, bhn, wos, wom, bf)


def kernel(node_features, edge_index, W_s, b_s, W_mp, b_mp, Wi_r, bi_r, Wh_r,
           Wi_z, bi_z, Wh_z, Wi_n, bi_n, Wh_n, bh_n, W_o, b_o, conv_k, conv_b):
    Wi_cat = jnp.concatenate([Wi_r, Wi_z, Wi_n], axis=1)
    bi_cat = jnp.concatenate([bi_r, bi_z, bi_n])
    p1 = W_mp[:H] @ Wi_cat
    p2 = W_mp[H:] @ Wi_cat
    ba = (b_mp @ Wi_cat + bi_cat).reshape(1, 3 * M)
    wh = jnp.concatenate([Wh_r, Wh_z, Wh_n], axis=1)
    ii = jnp.arange(H)[:, None]
    hh = jnp.arange(H)[None, :]
    kk = ii - hh + (TW - 1) // 2
    cmat = jnp.where((kk >= 0) & (kk < TW),
                     conv_k[jnp.clip(kk, 0, TW - 1), 0, 0], 0.0)
    wos = W_o[:H] @ cmat
    wom = W_o[H:] @ cmat
    bf = (b_o @ cmat + conv_b[0]).reshape(1, H)

    s_flat = _compute_s(node_features.reshape(T * N, D), W_s, b_s)
    s3 = s_flat.reshape(T, N, H)
    pad = jnp.zeros((EPAD - E,), jnp.int32)
    src2d = jnp.concatenate([edge_index[0], pad]).reshape(NC * NS, CPT, 1, CHUNK)
    dst2d = jnp.concatenate([edge_index[1], pad + N]).reshape(NC * NS, CPT, 1, CHUNK)
    idx4d = jnp.concatenate([src2d, dst2d], axis=2)
    zeros_block = jnp.zeros((ZCH, H), jnp.float32)
    p0, p1m = _compute_msgs(s3, idx4d, zeros_block)
    final, mem = _compute_gru(s3, p0, p1m, p1, p2, ba, wh,
                              bh_n.reshape(1, M), wos, wom, bf)
    return (final, mem)
